# baseline (speedup 1.0000x reference)
import jax
import jax.numpy as jnp
from jax.experimental import pallas as pl
from jax.experimental.pallas import tpu as pltpu

B = 128
N_ITERS = 3
W = 16
NEG = -1e30


def _pick_chunk(E):
    for c in (8000, 4000, 2000, 1000, 500, 320, 200, 100):
        if E % c == 0 and c % 8 == 0:
            return c
    return E


def _body(ebmin_ref, ebmax_ref, feat_ref, eb_ref, wih_ref, whh_ref, b_ref,
          out_ref, h_s, c_s, q_s, qs_s, m_s, l_s, acc_s):
    i = pl.program_id(0)
    j = pl.program_id(1)
    wi = pl.program_id(2)
    nchunks = pl.num_programs(1)
    D = feat_ref.shape[1]

    @pl.when(jnp.logical_and(i == 0, jnp.logical_and(j == 0, wi == 0)))
    def _init():
        h_s[...] = jnp.zeros_like(h_s)
        c_s[...] = jnp.zeros_like(c_s)
        qs_s[...] = jnp.zeros_like(qs_s)

    @pl.when(jnp.logical_and(j == 0, wi == 0))
    def _lstm():
        gates = jax.lax.dot_general(
            qs_s[...], wih_ref[...], (((1,), (1,)), ((), ())),
            preferred_element_type=jnp.float32)
        gates = gates + jax.lax.dot_general(
            h_s[...], whh_ref[...], (((1,), (1,)), ((), ())),
            preferred_element_type=jnp.float32)
        gates = gates + b_ref[...]
        ig = jax.nn.sigmoid(gates[:, 0 * D:1 * D])
        fg = jax.nn.sigmoid(gates[:, 1 * D:2 * D])
        gg = jnp.tanh(gates[:, 2 * D:3 * D])
        og = jax.nn.sigmoid(gates[:, 3 * D:4 * D])
        c_new = fg * c_s[...] + ig * gg
        h_new = og * jnp.tanh(c_new)
        c_s[...] = c_new
        h_s[...] = h_new
        q_s[...] = h_new
        m_s[...] = jnp.full_like(m_s, NEG)
        l_s[...] = jnp.zeros_like(l_s)
        acc_s[...] = jnp.zeros_like(acc_s)

    b0 = wi * W
    active = jnp.logical_and(ebmax_ref[j] >= b0, ebmin_ref[j] < b0 + W)

    @pl.when(active)
    def _chunk():
        feat_blk = feat_ref[...]
        eb = eb_ref[0]
        seg_ids = b0 + jax.lax.broadcasted_iota(jnp.int32, (W, 1), 0)
        maskT = eb == seg_ids

        pt = jax.lax.dot_general(
            q_s[pl.ds(b0, W), :], feat_blk, (((1,), (1,)), ((), ())),
            preferred_element_type=jnp.float32)
        pm = jnp.where(maskT, pt, NEG)
        m_chunk = jnp.max(pm, axis=1, keepdims=True)
        m_old = m_s[pl.ds(b0, W), :]
        m_new = jnp.maximum(m_old, m_chunk)
        scale = jnp.exp(m_old - m_new)
        w = jnp.where(maskT, jnp.exp(pt - m_new), 0.0)
        l_s[pl.ds(b0, W), :] = (l_s[pl.ds(b0, W), :] * scale
                                + jnp.sum(w, axis=1, keepdims=True))
        acc_s[pl.ds(b0, W), :] = (acc_s[pl.ds(b0, W), :] * scale
                                  + jax.lax.dot_general(
            w, feat_blk, (((1,), (0,)), ((), ())),
            preferred_element_type=jnp.float32))
        m_s[pl.ds(b0, W), :] = m_new

    @pl.when(jnp.logical_and(j == nchunks - 1, wi == pl.num_programs(2) - 1))
    def _finish():
        readout = acc_s[...] / (l_s[...] + 1e-8)
        qs_new = jnp.concatenate([q_s[...], readout], axis=1)
        qs_s[...] = qs_new
        out_ref[...] = qs_new


def kernel(feat, edge_batch, W_ih, W_hh, b_ih, b_hh):
    E, D = feat.shape
    CH = _pick_chunk(E)
    nchunks = E // CH
    eb32 = edge_batch.astype(jnp.int32)
    eb = eb32.reshape(nchunks, 1, CH)
    ebmin = eb32[::CH]
    ebmax = eb32[CH - 1::CH]
    bias = (b_ih + b_hh).reshape(1, 4 * D).astype(jnp.float32)

    grid = (N_ITERS, nchunks, B // W)
    out = pl.pallas_call(
        _body,
        grid_spec=pltpu.PrefetchScalarGridSpec(
            num_scalar_prefetch=2,
            grid=grid,
            in_specs=[
                pl.BlockSpec((CH, D), lambda i, j, wi, *_: (j, 0)),
                pl.BlockSpec((1, 1, CH), lambda i, j, wi, *_: (j, 0, 0)),
                pl.BlockSpec((4 * D, 2 * D), lambda i, j, wi, *_: (0, 0)),
                pl.BlockSpec((4 * D, D), lambda i, j, wi, *_: (0, 0)),
                pl.BlockSpec((1, 4 * D), lambda i, j, wi, *_: (0, 0)),
            ],
            out_specs=pl.BlockSpec((B, 2 * D), lambda i, j, wi, *_: (0, 0)),
            scratch_shapes=[
                pltpu.VMEM((B, D), jnp.float32),
                pltpu.VMEM((B, D), jnp.float32),
                pltpu.VMEM((B, D), jnp.float32),
                pltpu.VMEM((B, 2 * D), jnp.float32),
                pltpu.VMEM((B, 1), jnp.float32),
                pltpu.VMEM((B, 1), jnp.float32),
                pltpu.VMEM((B, D), jnp.float32),
            ],
        ),
        out_shape=jax.ShapeDtypeStruct((B, 2 * D), jnp.float32),
    )(ebmin, ebmax, feat, eb, W_ih, W_hh, bias)
    return out


# speedup vs baseline: 1.7587x; 1.7587x over previous
import jax
import jax.numpy as jnp
from jax.experimental import pallas as pl
from jax.experimental.pallas import tpu as pltpu

B = 128
N_ITERS = 3
W = 16
NEG = -1e30


def _pick_chunk(E):
    for c in (8000, 4000, 2000, 1000, 500, 320, 200, 100):
        if E % c == 0 and c % 8 == 0:
            return c
    return E


def _body(ebmin_ref, ebmax_ref, feat_ref, eb_ref, wih_ref, whh_ref, b_ref,
          out_ref, h_s, c_s, q_s, qs_s, m_s, l_s, acc_s):
    i = pl.program_id(0)
    j = pl.program_id(1)
    nchunks = pl.num_programs(1)
    D = feat_ref.shape[1]

    @pl.when(jnp.logical_and(i == 0, j == 0))
    def _init():
        h_s[...] = jnp.zeros_like(h_s)
        c_s[...] = jnp.zeros_like(c_s)
        qs_s[...] = jnp.zeros_like(qs_s)

    @pl.when(j == 0)
    def _lstm():
        gates = jax.lax.dot_general(
            qs_s[...], wih_ref[...], (((1,), (1,)), ((), ())),
            preferred_element_type=jnp.float32)
        gates = gates + jax.lax.dot_general(
            h_s[...], whh_ref[...], (((1,), (1,)), ((), ())),
            preferred_element_type=jnp.float32)
        gates = gates + b_ref[...]
        ig = jax.nn.sigmoid(gates[:, 0 * D:1 * D])
        fg = jax.nn.sigmoid(gates[:, 1 * D:2 * D])
        gg = jnp.tanh(gates[:, 2 * D:3 * D])
        og = jax.nn.sigmoid(gates[:, 3 * D:4 * D])
        c_new = fg * c_s[...] + ig * gg
        h_new = og * jnp.tanh(c_new)
        c_s[...] = c_new
        h_s[...] = h_new
        q_s[...] = h_new
        m_s[...] = jnp.full_like(m_s, NEG)
        l_s[...] = jnp.zeros_like(l_s)
        acc_s[...] = jnp.zeros_like(acc_s)

    feat_blk = feat_ref[...]
    eb = eb_ref[0]

    def _window(wi, carry):
        b0 = wi * W
        seg_ids = b0 + jax.lax.broadcasted_iota(jnp.int32, (W, 1), 0)
        maskT = eb == seg_ids

        pt = jax.lax.dot_general(
            q_s[pl.ds(b0, W), :], feat_blk, (((1,), (1,)), ((), ())),
            preferred_element_type=jnp.float32)
        pm = jnp.where(maskT, pt, NEG)
        m_chunk = jnp.max(pm, axis=1, keepdims=True)
        m_old = m_s[pl.ds(b0, W), :]
        m_new = jnp.maximum(m_old, m_chunk)
        scale = jnp.exp(m_old - m_new)
        w = jnp.where(maskT, jnp.exp(pt - m_new), 0.0)
        l_s[pl.ds(b0, W), :] = (l_s[pl.ds(b0, W), :] * scale
                                + jnp.sum(w, axis=1, keepdims=True))
        acc_s[pl.ds(b0, W), :] = (acc_s[pl.ds(b0, W), :] * scale
                                  + jax.lax.dot_general(
            w, feat_blk, (((1,), (0,)), ((), ())),
            preferred_element_type=jnp.float32))
        m_s[pl.ds(b0, W), :] = m_new
        return carry

    w_lo = ebmin_ref[j] // W
    w_hi = ebmax_ref[j] // W
    jax.lax.fori_loop(w_lo, w_hi + 1, _window, 0)

    @pl.when(j == nchunks - 1)
    def _finish():
        readout = acc_s[...] / (l_s[...] + 1e-8)
        qs_new = jnp.concatenate([q_s[...], readout], axis=1)
        qs_s[...] = qs_new
        out_ref[...] = qs_new


def kernel(feat, edge_batch, W_ih, W_hh, b_ih, b_hh):
    E, D = feat.shape
    CH = _pick_chunk(E)
    nchunks = E // CH
    eb32 = edge_batch.astype(jnp.int32)
    eb = eb32.reshape(nchunks, 1, CH)
    ebmin = eb32[::CH]
    ebmax = eb32[CH - 1::CH]
    bias = (b_ih + b_hh).reshape(1, 4 * D).astype(jnp.float32)

    grid = (N_ITERS, nchunks)
    out = pl.pallas_call(
        _body,
        grid_spec=pltpu.PrefetchScalarGridSpec(
            num_scalar_prefetch=2,
            grid=grid,
            in_specs=[
                pl.BlockSpec((CH, D), lambda i, j, *_: (j, 0)),
                pl.BlockSpec((1, 1, CH), lambda i, j, *_: (j, 0, 0)),
                pl.BlockSpec((4 * D, 2 * D), lambda i, j, *_: (0, 0)),
                pl.BlockSpec((4 * D, D), lambda i, j, *_: (0, 0)),
                pl.BlockSpec((1, 4 * D), lambda i, j, *_: (0, 0)),
            ],
            out_specs=pl.BlockSpec((B, 2 * D), lambda i, j, *_: (0, 0)),
            scratch_shapes=[
                pltpu.VMEM((B, D), jnp.float32),
                pltpu.VMEM((B, D), jnp.float32),
                pltpu.VMEM((B, D), jnp.float32),
                pltpu.VMEM((B, 2 * D), jnp.float32),
                pltpu.VMEM((B, 1), jnp.float32),
                pltpu.VMEM((B, 1), jnp.float32),
                pltpu.VMEM((B, D), jnp.float32),
            ],
        ),
        out_shape=jax.ShapeDtypeStruct((B, 2 * D), jnp.float32),
    )(ebmin, ebmax, feat, eb, W_ih, W_hh, bias)
    return out


# speedup vs baseline: 1.9264x; 1.0954x over previous
import jax
import jax.numpy as jnp
from jax.experimental import pallas as pl
from jax.experimental.pallas import tpu as pltpu

B = 128
N_ITERS = 3
W = 16
NEG = -1e30


def _pick_chunk(E):
    for c in (16000, 8000, 4000, 2000, 1000, 500, 320, 200, 100):
        if E % c == 0 and c % 8 == 0:
            return c
    return E


def _body(ebmin_ref, ebmax_ref, feat_ref, eb_ref, wih_ref, whh_ref, b_ref,
          out_ref, h_s, c_s, q_s, qs_s, m_s, l_s, acc_s):
    i = pl.program_id(0)
    j = pl.program_id(1)
    nchunks = pl.num_programs(1)
    D = feat_ref.shape[1]

    @pl.when(jnp.logical_and(i == 0, j == 0))
    def _init():
        h_s[...] = jnp.zeros_like(h_s)
        c_s[...] = jnp.zeros_like(c_s)
        qs_s[...] = jnp.zeros_like(qs_s)

    @pl.when(j == 0)
    def _lstm():
        gates = jax.lax.dot_general(
            qs_s[...], wih_ref[...], (((1,), (1,)), ((), ())),
            preferred_element_type=jnp.float32)
        gates = gates + jax.lax.dot_general(
            h_s[...], whh_ref[...], (((1,), (1,)), ((), ())),
            preferred_element_type=jnp.float32)
        gates = gates + b_ref[...]
        ig = jax.nn.sigmoid(gates[:, 0 * D:1 * D])
        fg = jax.nn.sigmoid(gates[:, 1 * D:2 * D])
        gg = jnp.tanh(gates[:, 2 * D:3 * D])
        og = jax.nn.sigmoid(gates[:, 3 * D:4 * D])
        c_new = fg * c_s[...] + ig * gg
        h_new = og * jnp.tanh(c_new)
        c_s[...] = c_new
        h_s[...] = h_new
        q_s[...] = h_new
        m_s[...] = jnp.full_like(m_s, NEG)
        l_s[...] = jnp.zeros_like(l_s)
        acc_s[...] = jnp.zeros_like(acc_s)

    feat_blk = feat_ref[...]
    eb = eb_ref[0]

    def _window(wi, carry):
        b0 = wi * W
        seg_ids = b0 + jax.lax.broadcasted_iota(jnp.int32, (W, 1), 0)
        maskT = eb == seg_ids

        pt = jax.lax.dot_general(
            q_s[pl.ds(b0, W), :], feat_blk, (((1,), (1,)), ((), ())),
            preferred_element_type=jnp.float32)
        pm = jnp.where(maskT, pt, NEG)
        m_chunk = jnp.max(pm, axis=1, keepdims=True)
        m_old = m_s[pl.ds(b0, W), :]
        m_new = jnp.maximum(m_old, m_chunk)
        scale = jnp.exp(m_old - m_new)
        w = jnp.where(maskT, jnp.exp(pt - m_new), 0.0)
        l_s[pl.ds(b0, W), :] = (l_s[pl.ds(b0, W), :] * scale
                                + jnp.sum(w, axis=1, keepdims=True))
        acc_s[pl.ds(b0, W), :] = (acc_s[pl.ds(b0, W), :] * scale
                                  + jax.lax.dot_general(
            w, feat_blk, (((1,), (0,)), ((), ())),
            preferred_element_type=jnp.float32))
        m_s[pl.ds(b0, W), :] = m_new
        return carry

    w_lo = ebmin_ref[j] // W
    w_hi = ebmax_ref[j] // W
    jax.lax.fori_loop(w_lo, w_hi + 1, _window, 0)

    @pl.when(j == nchunks - 1)
    def _finish():
        readout = acc_s[...] / (l_s[...] + 1e-8)
        qs_new = jnp.concatenate([q_s[...], readout], axis=1)
        qs_s[...] = qs_new
        out_ref[...] = qs_new


def kernel(feat, edge_batch, W_ih, W_hh, b_ih, b_hh):
    E, D = feat.shape
    CH = _pick_chunk(E)
    nchunks = E // CH
    eb32 = edge_batch.astype(jnp.int32)
    eb = eb32.reshape(nchunks, 1, CH)
    ebmin = eb32[::CH]
    ebmax = eb32[CH - 1::CH]
    bias = (b_ih + b_hh).reshape(1, 4 * D).astype(jnp.float32)

    grid = (N_ITERS, nchunks)
    out = pl.pallas_call(
        _body,
        grid_spec=pltpu.PrefetchScalarGridSpec(
            num_scalar_prefetch=2,
            grid=grid,
            in_specs=[
                pl.BlockSpec((CH, D), lambda i, j, *_: (j, 0)),
                pl.BlockSpec((1, 1, CH), lambda i, j, *_: (j, 0, 0)),
                pl.BlockSpec((4 * D, 2 * D), lambda i, j, *_: (0, 0)),
                pl.BlockSpec((4 * D, D), lambda i, j, *_: (0, 0)),
                pl.BlockSpec((1, 4 * D), lambda i, j, *_: (0, 0)),
            ],
            out_specs=pl.BlockSpec((B, 2 * D), lambda i, j, *_: (0, 0)),
            scratch_shapes=[
                pltpu.VMEM((B, D), jnp.float32),
                pltpu.VMEM((B, D), jnp.float32),
                pltpu.VMEM((B, D), jnp.float32),
                pltpu.VMEM((B, 2 * D), jnp.float32),
                pltpu.VMEM((B, 1), jnp.float32),
                pltpu.VMEM((B, 1), jnp.float32),
                pltpu.VMEM((B, D), jnp.float32),
            ],
        ),
        out_shape=jax.ShapeDtypeStruct((B, 2 * D), jnp.float32),
    )(ebmin, ebmax, feat, eb, W_ih, W_hh, bias)
    return out
